# baseline (speedup 1.0000x reference)
import functools

import jax
import jax.numpy as jnp
from jax import lax
from jax.experimental import pallas as pl
from jax.experimental.pallas import tpu as pltpu
from jax.experimental.pallas import tpu_sc as plsc

N = 100000
E = 1600000
H = 32

NC = 2
NS = 16
NW = NC * NS

C = 128
NCHUNK = E // C
DSUB = 25
NBIG = E // (DSUB * C)
_mesh = plsc.VectorSubcoreMesh(core_axis_name="c", subcore_axis_name="s",
                               num_cores=NC, num_subcores=NS)
_sc_params = pltpu.CompilerParams(use_tc_tiling_on_sc=False)


@functools.partial(
    pl.kernel,
    out_type=(jax.ShapeDtypeStruct((N, 16), jnp.float32),
              jax.ShapeDtypeStruct((N, 16), jnp.float32)),
    mesh=_mesh,
    compiler_params=_sc_params,
    scratch_types=dict(
        hist=pltpu.VMEM_SHARED((N, 16), jnp.float32),
        zbuf=pltpu.VMEM((400, 16), jnp.float32),
        ones=pltpu.VMEM((C, 16), jnp.float32),
        dbuf=pltpu.VMEM((2, DSUB, C), jnp.int32),
        sem_ld0=pltpu.SemaphoreType.DMA,
        sem_ld1=pltpu.SemaphoreType.DMA,
        sem_sc0=pltpu.SemaphoreType.DMA,
        sem_sc1=pltpu.SemaphoreType.DMA,
    ),
)
def _deg_kernel(dst_hbm, cnt0_hbm, cnt1_hbm, hist, zbuf, ones, dbuf,
                sem_ld0, sem_ld1, sem_sc0, sem_sc1):
    c = lax.axis_index("c")
    s = lax.axis_index("s")
    wid = s * NC + c

    def fz(r, _):
        zbuf[r, pl.ds(0, 16)] = jnp.zeros((16,), jnp.float32)
        return 0
    lax.fori_loop(0, 400, fz, 0)

    def fo(r, _):
        ones[r, pl.ds(0, 16)] = jnp.ones((16,), jnp.float32)
        return 0
    lax.fori_loop(0, C, fo, 0)

    nz = jnp.where(s < NS - 1, 16, 10)

    def z(k, _):
        pltpu.sync_copy(zbuf, hist.at[pl.ds(s * 6400 + k * 400, 400)])
        return 0
    lax.fori_loop(0, nz, z, 0)

    plsc.subcore_barrier()

    sld = (sem_ld0, sem_ld1)
    ssc = (sem_sc0, sem_sc1)
    nb = (NBIG - wid + NW - 1) // NW

    def base_of(k):
        return (wid + k * NW) * (DSUB * C)

    def issue_loads(k, b):
        base = base_of(k)
        for j in range(DSUB):
            pltpu.async_copy(dst_hbm.at[pl.ds(base + j * C, C)],
                             dbuf.at[b, j], sld[b])

    def wait_loads(k, b):
        base = base_of(k)
        for j in range(DSUB):
            pltpu.make_async_copy(dst_hbm.at[pl.ds(base + j * C, C)],
                                  dbuf.at[b, j], sld[b]).wait()

    def issue_scatters(b):
        for j in range(DSUB):
            pltpu.async_copy(ones, hist.at[dbuf.at[b, j]], ssc[b], add=True)

    def drain_scatters(b):
        for j in range(DSUB):
            pltpu.make_async_copy(ones, hist.at[dbuf.at[b, j]], ssc[b]).wait()

    def halfstep(k, b):
        wait_loads(k, b)

        @pl.when(k >= 1)
        def _():
            drain_scatters(1 - b)
        issue_scatters(b)

        @pl.when(k + 1 < nb)
        def _():
            issue_loads(k + 1, 1 - b)

    issue_loads(0, 0)

    def pairstep(p, _):
        k0 = p * 2
        halfstep(k0, 0)
        k1 = k0 + 1

        @pl.when(k1 < nb)
        def _():
            halfstep(k1, 1)
        return 0
    lax.fori_loop(0, (nb + 1) // 2, pairstep, 0)

    @pl.when(nb % 2 == 1)
    def _():
        drain_scatters(0)

    @pl.when(nb % 2 == 0)
    def _():
        drain_scatters(1)

    plsc.subcore_barrier()

    @pl.when(s < NS - 1)
    def _():
        @pl.when(c == 0)
        def _():
            pltpu.sync_copy(hist.at[pl.ds(s * 6400, 6400)],
                            cnt0_hbm.at[pl.ds(s * 6400, 6400)])

        @pl.when(c == 1)
        def _():
            pltpu.sync_copy(hist.at[pl.ds(s * 6400, 6400)],
                            cnt1_hbm.at[pl.ds(s * 6400, 6400)])

    @pl.when(s == NS - 1)
    def _():
        @pl.when(c == 0)
        def _():
            pltpu.sync_copy(hist.at[pl.ds(96000, 4000)],
                            cnt0_hbm.at[pl.ds(96000, 4000)])

        @pl.when(c == 1)
        def _():
            pltpu.sync_copy(hist.at[pl.ds(96000, 4000)],
                            cnt1_hbm.at[pl.ds(96000, 4000)])


G = 4
GC = G * C
NGROUP = E // GC


@functools.partial(
    pl.kernel,
    out_type=(jax.ShapeDtypeStruct((N, 16), jnp.float32),
              jax.ShapeDtypeStruct((N, 16), jnp.float32)),
    mesh=_mesh,
    compiler_params=_sc_params,
    scratch_types=dict(
        acc=pltpu.VMEM_SHARED((N, 16), jnp.float32),
        zbuf=pltpu.VMEM((100, 16), jnp.float32),
        sbuf=pltpu.VMEM((2, G, C), jnp.int32),
        dbuf=pltpu.VMEM((2, G, C), jnp.int32),
        ibuf=pltpu.VMEM((2, G, C), jnp.int32),
        rows=pltpu.VMEM((2, G, C, 16), jnp.float32),
        sem_ld0=pltpu.SemaphoreType.DMA,
        sem_ld1=pltpu.SemaphoreType.DMA,
        sem_g0=pltpu.SemaphoreType.DMA,
        sem_g1=pltpu.SemaphoreType.DMA,
        sem_sc0=pltpu.SemaphoreType.DMA,
        sem_sc1=pltpu.SemaphoreType.DMA,
    ),
)
def _scatter_kernel(src_hbm, dst_hbm, ga_hbm, gb_hbm, outa_hbm, outb_hbm,
                    acc, zbuf, sbuf, dbuf, ibuf, rows,
                    sem_ld0, sem_ld1, sem_g0, sem_g1, sem_sc0, sem_sc1):
    c = lax.axis_index("c")
    s = lax.axis_index("s")

    def fz(r, _):
        zbuf[r, pl.ds(0, 16)] = jnp.zeros((16,), jnp.float32)
        return 0
    lax.fori_loop(0, 100, fz, 0)

    nz = jnp.where(s < NS - 1, 64, 40)

    def z(k, _):
        pltpu.sync_copy(zbuf, acc.at[pl.ds(s * 6400 + k * 100, 100)])
        return 0
    lax.fori_loop(0, nz, z, 0)

    plsc.subcore_barrier()

    sld = (sem_ld0, sem_ld1)
    sg = (sem_g0, sem_g1)
    ssc = (sem_sc0, sem_sc1)
    nk = (NGROUP - s + NS - 1) // NS

    def run(tbl_hbm):
        def base_of(k):
            return (s + k * NS) * GC

        def issue_loads(k, b):
            base = base_of(k)
            for j in range(G):
                pltpu.async_copy(src_hbm.at[pl.ds(base + j * C, C)],
                                 sbuf.at[b, j], sld[b])
                pltpu.async_copy(dst_hbm.at[pl.ds(base + j * C, C)],
                                 dbuf.at[b, j], sld[b])

        def wait_loads(k, b):
            base = base_of(k)
            for j in range(G):
                pltpu.make_async_copy(src_hbm.at[pl.ds(base + j * C, C)],
                                      sbuf.at[b, j], sld[b]).wait()
                pltpu.make_async_copy(dst_hbm.at[pl.ds(base + j * C, C)],
                                      dbuf.at[b, j], sld[b]).wait()

        def copy_ibuf(b):
            for j in range(G):
                def mk(m, _):
                    ibuf[b, j, pl.ds(m * 16, 16)] = dbuf[b, j, pl.ds(m * 16, 16)]
                    return 0
                lax.fori_loop(0, C // 16, mk, 0)

        def issue_gathers(b):
            for j in range(G):
                pltpu.async_copy(tbl_hbm.at[sbuf.at[b, j]], rows.at[b, j], sg[b])

        def drain_gathers(b):
            for j in range(G):
                pltpu.make_async_copy(tbl_hbm.at[sbuf.at[b, j]],
                                      rows.at[b, j], sg[b]).wait()

        def issue_scatters(b):
            for j in range(G):
                pltpu.async_copy(rows.at[b, j], acc.at[ibuf.at[b, j]],
                                 ssc[b], add=True)

        def drain_scatters(b):
            for j in range(G):
                pltpu.make_async_copy(rows.at[b, j], acc.at[ibuf.at[b, j]],
                                      ssc[b]).wait()

        def halfstep(k, b):
            wait_loads(k, b)

            @pl.when(k >= 2)
            def _():
                drain_scatters(b)
            copy_ibuf(b)
            issue_gathers(b)

            @pl.when(k >= 1)
            def _():
                drain_gathers(1 - b)

            @pl.when(k + 1 < nk)
            def _():
                issue_loads(k + 1, 1 - b)

            @pl.when(k >= 1)
            def _():
                issue_scatters(1 - b)

        issue_loads(0, 0)

        def pair(p, _):
            k0 = p * 2
            halfstep(k0, 0)
            k1 = k0 + 1

            @pl.when(k1 < nk)
            def _():
                halfstep(k1, 1)
            return 0
        lax.fori_loop(0, (nk + 1) // 2, pair, 0)

        @pl.when(nk % 2 == 1)
        def _():
            drain_gathers(0)
            issue_scatters(0)
            drain_scatters(1)
            drain_scatters(0)

        @pl.when(nk % 2 == 0)
        def _():
            drain_gathers(1)
            issue_scatters(1)
            drain_scatters(0)
            drain_scatters(1)

    @pl.when(c == 0)
    def _():
        run(ga_hbm)

    @pl.when(c == 1)
    def _():
        run(gb_hbm)

    plsc.subcore_barrier()

    @pl.when(s < NS - 1)
    def _():
        @pl.when(c == 0)
        def _():
            pltpu.sync_copy(acc.at[pl.ds(s * 6400, 6400)],
                            outa_hbm.at[pl.ds(s * 6400, 6400)])

        @pl.when(c == 1)
        def _():
            pltpu.sync_copy(acc.at[pl.ds(s * 6400, 6400)],
                            outb_hbm.at[pl.ds(s * 6400, 6400)])

    @pl.when(s == NS - 1)
    def _():
        @pl.when(c == 0)
        def _():
            pltpu.sync_copy(acc.at[pl.ds(96000, 4000)],
                            outa_hbm.at[pl.ds(96000, 4000)])

        @pl.when(c == 1)
        def _():
            pltpu.sync_copy(acc.at[pl.ds(96000, 4000)],
                            outb_hbm.at[pl.ds(96000, 4000)])


EG = 2
EGC = EG * C
NEGROUP = E // EGC


@functools.partial(
    pl.kernel,
    out_type=jax.ShapeDtypeStruct((E, H), jnp.float32),
    mesh=_mesh,
    compiler_params=_sc_params,
    scratch_types=dict(
        sbuf=pltpu.VMEM((2, EG, C), jnp.int32),
        dbuf=pltpu.VMEM((2, EG, C), jnp.int32),
        prow=pltpu.VMEM((2, EG, C, H), jnp.float32),
        qrow=pltpu.VMEM((2, EG, C, H), jnp.float32),
        sem_ld0=pltpu.SemaphoreType.DMA,
        sem_ld1=pltpu.SemaphoreType.DMA,
        sem_g0=pltpu.SemaphoreType.DMA,
        sem_g1=pltpu.SemaphoreType.DMA,
        sem_st0=pltpu.SemaphoreType.DMA,
        sem_st1=pltpu.SemaphoreType.DMA,
    ),
)
def _edge_gather_kernel(src_hbm, dst_hbm, p_hbm, q_hbm, eo_hbm,
                        sbuf, dbuf, prow, qrow,
                        sem_ld0, sem_ld1, sem_g0, sem_g1, sem_st0, sem_st1):
    c = lax.axis_index("c")
    s = lax.axis_index("s")
    wid = s * NC + c
    sld = (sem_ld0, sem_ld1)
    sg = (sem_g0, sem_g1)
    sst = (sem_st0, sem_st1)
    nk = (NEGROUP - wid + NW - 1) // NW

    def base_of(k):
        return (wid + k * NW) * EGC

    def issue_loads(k, b):
        base = base_of(k)
        for j in range(EG):
            pltpu.async_copy(src_hbm.at[pl.ds(base + j * C, C)],
                             sbuf.at[b, j], sld[b])
            pltpu.async_copy(dst_hbm.at[pl.ds(base + j * C, C)],
                             dbuf.at[b, j], sld[b])

    def wait_loads(k, b):
        base = base_of(k)
        for j in range(EG):
            pltpu.make_async_copy(src_hbm.at[pl.ds(base + j * C, C)],
                                  sbuf.at[b, j], sld[b]).wait()
            pltpu.make_async_copy(dst_hbm.at[pl.ds(base + j * C, C)],
                                  dbuf.at[b, j], sld[b]).wait()

    def issue_gathers(b):
        for j in range(EG):
            pltpu.async_copy(p_hbm.at[sbuf.at[b, j]], prow.at[b, j], sg[b])
            pltpu.async_copy(q_hbm.at[dbuf.at[b, j]], qrow.at[b, j], sg[b])

    def drain_gathers(b):
        for j in range(EG):
            pltpu.make_async_copy(p_hbm.at[sbuf.at[b, j]],
                                  prow.at[b, j], sg[b]).wait()
            pltpu.make_async_copy(q_hbm.at[dbuf.at[b, j]],
                                  qrow.at[b, j], sg[b]).wait()

    def add_rows(b):
        for j in range(EG):
            def add(r, _):
                prow[b, j, r, pl.ds(0, 16)] = (
                    prow[b, j, r, pl.ds(0, 16)] + qrow[b, j, r, pl.ds(0, 16)])
                prow[b, j, r, pl.ds(16, 16)] = (
                    prow[b, j, r, pl.ds(16, 16)] + qrow[b, j, r, pl.ds(16, 16)])
                return 0
            lax.fori_loop(0, C, add, 0, unroll=4)

    def issue_stores(k, b):
        base = base_of(k)
        for j in range(EG):
            pltpu.async_copy(prow.at[b, j],
                             eo_hbm.at[pl.ds(base + j * C, C)], sst[b])

    def drain_stores(k, b):
        base = base_of(k)
        for j in range(EG):
            pltpu.make_async_copy(prow.at[b, j],
                                  eo_hbm.at[pl.ds(base + j * C, C)],
                                  sst[b]).wait()

    def halfstep(k, b):
        wait_loads(k, b)

        @pl.when(k >= 2)
        def _():
            drain_stores(k - 2, b)
        issue_gathers(b)

        @pl.when(k >= 1)
        def _():
            drain_gathers(1 - b)
            add_rows(1 - b)
            issue_stores(k - 1, 1 - b)

        @pl.when(k + 1 < nk)
        def _():
            issue_loads(k + 1, 1 - b)

    issue_loads(0, 0)

    def pair(p, _):
        k0 = p * 2
        halfstep(k0, 0)
        k1 = k0 + 1

        @pl.when(k1 < nk)
        def _():
            halfstep(k1, 1)
        return 0
    lax.fori_loop(0, (nk + 1) // 2, pair, 0)

    @pl.when(nk % 2 == 1)
    def _():
        drain_gathers(0)
        add_rows(0)
        issue_stores(nk - 1, 0)
        drain_stores(nk - 2, 1)
        drain_stores(nk - 1, 0)

    @pl.when(nk % 2 == 0)
    def _():
        drain_gathers(1)
        add_rows(1)
        issue_stores(nk - 1, 1)
        drain_stores(nk - 2, 0)
        drain_stores(nk - 1, 1)


_BN = 2048


def _mm(w, a):
    return lax.dot_general(w, a, (((1,), (0,)), ((), ())),
                           preferred_element_type=jnp.float32)


def _node_a_body(xt_ref, cnt0t_ref, cnt1t_ref, w1_ref, b1_ref, w2_ref,
                 b2_ref, wg1_ref, g1_ref, dinv_ref):
    h = jnp.maximum(_mm(w1_ref[...], xt_ref[...]) + b1_ref[...][:, None], 0.0)
    h = _mm(w2_ref[...], h) + b2_ref[...][:, None]
    deg = cnt0t_ref[0:1, :] + cnt1t_ref[0:1, :] + 1.0
    dinv = lax.rsqrt(deg)
    dinv_ref[...] = dinv
    g1_ref[...] = _mm(wg1_ref[...], h) * dinv


def _node_b_body(tmp1at_ref, tmp1bt_ref, g1_ref, dinv_ref, bg1_ref, wg2_ref,
                 g2_ref):
    dinv = dinv_ref[...]
    tmp1t = jnp.concatenate([tmp1at_ref[...], tmp1bt_ref[...]], axis=0)
    h1 = jnp.maximum(dinv * (tmp1t + g1_ref[...])
                     + bg1_ref[...][:, None], 0.0)
    g2_ref[...] = _mm(wg2_ref[...], h1) * dinv


def _node_c_body(tmp2at_ref, tmp2bt_ref, g2_ref, dinv_ref, bg2_ref, as_ref,
                 ad_ref, h_ref, p_ref, q_ref):
    tmp2t = jnp.concatenate([tmp2at_ref[...], tmp2bt_ref[...]], axis=0)
    h = jnp.maximum(dinv_ref[...] * (tmp2t + g2_ref[...])
                    + bg2_ref[...][:, None], 0.0)
    h_ref[...] = h
    p_ref[...] = _mm(as_ref[...], h)
    q_ref[...] = _mm(ad_ref[...], h)


_BQ = 3200
E4 = E // 4


def _edge_d_body(eoq_ref, ea12_ref, biga_ref, be1q_ref, bdw_ref, be2q_ref,
                 et_ref):
    z = eoq_ref[...] + lax.dot_general(
        ea12_ref[...], biga_ref[...], (((0,), (1,)), ((), ())),
        preferred_element_type=jnp.float32)
    t = jnp.maximum(z + be1q_ref[...][None, :], 0.0)
    eq = lax.dot_general(t, bdw_ref[...], (((1,), (0,)), ((), ())),
                         preferred_element_type=jnp.float32)
    eq = eq + be2q_ref[...][None, :]
    et_ref[...] = jnp.transpose(
        jnp.reshape(eq, (_BQ, 4, H)), (2, 0, 1)).reshape(H, 4 * _BQ)


def _col_spec(rows, b):
    return pl.BlockSpec((rows, b), lambda i: (0, i))


def _full(shape):
    return pl.BlockSpec(shape, lambda i: tuple(0 for _ in shape))


def kernel(x, edge_index, edge_attr, W1, b1, W2, b2, Wg1, bg1, Wg2, bg2,
           We1, be1, We2, be2):
    src = edge_index[0]
    dst = edge_index[1]

    cnt0, cnt1 = _deg_kernel(dst)

    n_grid = pl.cdiv(N, _BN)

    g1t, dinv = pl.pallas_call(
        _node_a_body,
        grid=(n_grid,),
        in_specs=[_col_spec(2, _BN), _col_spec(16, _BN), _col_spec(16, _BN),
                  _full((H, 2)), _full((H,)), _full((H, H)), _full((H,)),
                  _full((H, H))],
        out_specs=[_col_spec(H, _BN), _col_spec(1, _BN)],
        out_shape=[jax.ShapeDtypeStruct((H, N), jnp.float32),
                   jax.ShapeDtypeStruct((1, N), jnp.float32)],
    )(x.T, cnt0.T, cnt1.T, W1, b1, W2, b2, Wg1)

    tmp1a, tmp1b = _scatter_kernel(src, dst,
                                   jnp.transpose(g1t[:16]),
                                   jnp.transpose(g1t[16:]))

    g2t = pl.pallas_call(
        _node_b_body,
        grid=(n_grid,),
        in_specs=[_col_spec(16, _BN), _col_spec(16, _BN), _col_spec(H, _BN),
                  _col_spec(1, _BN), _full((H,)), _full((H, H))],
        out_specs=_col_spec(H, _BN),
        out_shape=jax.ShapeDtypeStruct((H, N), jnp.float32),
    )(tmp1a.T, tmp1b.T, g1t, dinv, bg1, Wg2)

    tmp2a, tmp2b = _scatter_kernel(src, dst,
                                   jnp.transpose(g2t[:16]),
                                   jnp.transpose(g2t[16:]))

    As = We1[:, :H]
    Ad = We1[:, H:2 * H]
    Aattr = We1[:, 2 * H:]

    ht, Pt, Qt = pl.pallas_call(
        _node_c_body,
        grid=(n_grid,),
        in_specs=[_col_spec(16, _BN), _col_spec(16, _BN), _col_spec(H, _BN),
                  _col_spec(1, _BN), _full((H,)), _full((H, H)),
                  _full((H, H))],
        out_specs=[_col_spec(H, _BN)] * 3,
        out_shape=[jax.ShapeDtypeStruct((H, N), jnp.float32)] * 3,
    )(tmp2a.T, tmp2b.T, g2t, dinv, bg2, As, Ad)

    eo = _edge_gather_kernel(src, dst, Pt.T, Qt.T)

    eoq = jnp.reshape(eo, (E4, 128))
    eye4 = jnp.eye(4, dtype=jnp.float32)
    BigA = jnp.kron(eye4, Aattr)
    BDW = jnp.kron(eye4, We2.T)
    be1q = jnp.tile(be1, 4)
    be2q = jnp.tile(be2, 4)
    ea12 = jnp.transpose(jnp.reshape(edge_attr.T, (3, E4, 4)),
                         (2, 0, 1)).reshape(12, E4)

    et = pl.pallas_call(
        _edge_d_body,
        grid=(E4 // _BQ,),
        in_specs=[pl.BlockSpec((_BQ, 128), lambda i: (i, 0)),
                  pl.BlockSpec((12, _BQ), lambda i: (0, i)),
                  _full((128, 12)), _full((128,)), _full((128, 128)),
                  _full((128,))],
        out_specs=pl.BlockSpec((H, 4 * _BQ), lambda i: (0, i)),
        out_shape=jax.ShapeDtypeStruct((H, E), jnp.float32),
    )(eoq, ea12, BigA, be1q, BDW, be2q)

    return ht.T, et.T


# speedup vs baseline: 2.3926x; 2.3926x over previous
import functools

import jax
import jax.numpy as jnp
from jax import lax
from jax.experimental import pallas as pl
from jax.experimental.pallas import tpu as pltpu
from jax.experimental.pallas import tpu_sc as plsc

N = 100000
E = 1600000
H = 32

NC = 2
NS = 16
NW = NC * NS

C = 128
NCHUNK = E // C
DSUB = 25
NBIG = E // (DSUB * C)
_mesh = plsc.VectorSubcoreMesh(core_axis_name="c", subcore_axis_name="s",
                               num_cores=NC, num_subcores=NS)
_sc_params = pltpu.CompilerParams(use_tc_tiling_on_sc=False)


@functools.partial(
    pl.kernel,
    out_type=(jax.ShapeDtypeStruct((N, 16), jnp.float32),
              jax.ShapeDtypeStruct((N, 16), jnp.float32)),
    mesh=_mesh,
    compiler_params=_sc_params,
    scratch_types=dict(
        hist=pltpu.VMEM_SHARED((N, 16), jnp.float32),
        zbuf=pltpu.VMEM((400, 16), jnp.float32),
        ones=pltpu.VMEM((C, 16), jnp.float32),
        dbuf=pltpu.VMEM((2, DSUB, C), jnp.int32),
        sem_ld0=pltpu.SemaphoreType.DMA,
        sem_ld1=pltpu.SemaphoreType.DMA,
        sem_sc0=pltpu.SemaphoreType.DMA,
        sem_sc1=pltpu.SemaphoreType.DMA,
    ),
)
def _deg_kernel(dst_hbm, cnt0_hbm, cnt1_hbm, hist, zbuf, ones, dbuf,
                sem_ld0, sem_ld1, sem_sc0, sem_sc1):
    c = lax.axis_index("c")
    s = lax.axis_index("s")
    wid = s * NC + c

    def fz(r, _):
        zbuf[r, pl.ds(0, 16)] = jnp.zeros((16,), jnp.float32)
        return 0
    lax.fori_loop(0, 400, fz, 0)

    def fo(r, _):
        ones[r, pl.ds(0, 16)] = jnp.ones((16,), jnp.float32)
        return 0
    lax.fori_loop(0, C, fo, 0)

    nz = jnp.where(s < NS - 1, 16, 10)

    def z(k, _):
        pltpu.sync_copy(zbuf, hist.at[pl.ds(s * 6400 + k * 400, 400)])
        return 0
    lax.fori_loop(0, nz, z, 0)

    plsc.subcore_barrier()

    sld = (sem_ld0, sem_ld1)
    ssc = (sem_sc0, sem_sc1)
    nb = (NBIG - wid + NW - 1) // NW

    def base_of(k):
        return (wid + k * NW) * (DSUB * C)

    def issue_loads(k, b):
        base = base_of(k)
        for j in range(DSUB):
            pltpu.async_copy(dst_hbm.at[pl.ds(base + j * C, C)],
                             dbuf.at[b, j], sld[b])

    def wait_loads(k, b):
        base = base_of(k)
        for j in range(DSUB):
            pltpu.make_async_copy(dst_hbm.at[pl.ds(base + j * C, C)],
                                  dbuf.at[b, j], sld[b]).wait()

    def issue_scatters(b):
        for j in range(DSUB):
            pltpu.async_copy(ones, hist.at[dbuf.at[b, j]], ssc[b], add=True)

    def drain_scatters(b):
        for j in range(DSUB):
            pltpu.make_async_copy(ones, hist.at[dbuf.at[b, j]], ssc[b]).wait()

    def halfstep(k, b):
        wait_loads(k, b)

        @pl.when(k >= 1)
        def _():
            drain_scatters(1 - b)
        issue_scatters(b)

        @pl.when(k + 1 < nb)
        def _():
            issue_loads(k + 1, 1 - b)

    issue_loads(0, 0)

    def pairstep(p, _):
        k0 = p * 2
        halfstep(k0, 0)
        k1 = k0 + 1

        @pl.when(k1 < nb)
        def _():
            halfstep(k1, 1)
        return 0
    lax.fori_loop(0, (nb + 1) // 2, pairstep, 0)

    @pl.when(nb % 2 == 1)
    def _():
        drain_scatters(0)

    @pl.when(nb % 2 == 0)
    def _():
        drain_scatters(1)

    plsc.subcore_barrier()

    @pl.when(s < NS - 1)
    def _():
        @pl.when(c == 0)
        def _():
            pltpu.sync_copy(hist.at[pl.ds(s * 6400, 6400)],
                            cnt0_hbm.at[pl.ds(s * 6400, 6400)])

        @pl.when(c == 1)
        def _():
            pltpu.sync_copy(hist.at[pl.ds(s * 6400, 6400)],
                            cnt1_hbm.at[pl.ds(s * 6400, 6400)])

    @pl.when(s == NS - 1)
    def _():
        @pl.when(c == 0)
        def _():
            pltpu.sync_copy(hist.at[pl.ds(96000, 4000)],
                            cnt0_hbm.at[pl.ds(96000, 4000)])

        @pl.when(c == 1)
        def _():
            pltpu.sync_copy(hist.at[pl.ds(96000, 4000)],
                            cnt1_hbm.at[pl.ds(96000, 4000)])


G = 4
GC = G * C
NGROUP = E // GC


@functools.partial(
    pl.kernel,
    out_type=(jax.ShapeDtypeStruct((N, 16), jnp.float32),
              jax.ShapeDtypeStruct((N, 16), jnp.float32)),
    mesh=_mesh,
    compiler_params=_sc_params,
    scratch_types=dict(
        acc=pltpu.VMEM_SHARED((N, 16), jnp.float32),
        zbuf=pltpu.VMEM((100, 16), jnp.float32),
        sbuf=pltpu.VMEM((2, G, C), jnp.int32),
        dbuf=pltpu.VMEM((2, G, C), jnp.int32),
        ibuf=pltpu.VMEM((2, G, C), jnp.int32),
        rows=pltpu.VMEM((2, G, C, 16), jnp.float32),
        sem_ld0=pltpu.SemaphoreType.DMA,
        sem_ld1=pltpu.SemaphoreType.DMA,
        sem_g0=pltpu.SemaphoreType.DMA,
        sem_g1=pltpu.SemaphoreType.DMA,
        sem_sc0=pltpu.SemaphoreType.DMA,
        sem_sc1=pltpu.SemaphoreType.DMA,
    ),
)
def _scatter_kernel(src_hbm, dst_hbm, ga_hbm, gb_hbm, outa_hbm, outb_hbm,
                    acc, zbuf, sbuf, dbuf, ibuf, rows,
                    sem_ld0, sem_ld1, sem_g0, sem_g1, sem_sc0, sem_sc1):
    c = lax.axis_index("c")
    s = lax.axis_index("s")

    def fz(r, _):
        zbuf[r, pl.ds(0, 16)] = jnp.zeros((16,), jnp.float32)
        return 0
    lax.fori_loop(0, 100, fz, 0)

    nz = jnp.where(s < NS - 1, 64, 40)

    def z(k, _):
        pltpu.sync_copy(zbuf, acc.at[pl.ds(s * 6400 + k * 100, 100)])
        return 0
    lax.fori_loop(0, nz, z, 0)

    plsc.subcore_barrier()

    sld = (sem_ld0, sem_ld1)
    sg = (sem_g0, sem_g1)
    ssc = (sem_sc0, sem_sc1)
    nk = (NGROUP - s + NS - 1) // NS

    def run(tbl_hbm):
        def base_of(k):
            return (s + k * NS) * GC

        def issue_loads(k, b):
            base = base_of(k)
            for j in range(G):
                pltpu.async_copy(src_hbm.at[pl.ds(base + j * C, C)],
                                 sbuf.at[b, j], sld[b])
                pltpu.async_copy(dst_hbm.at[pl.ds(base + j * C, C)],
                                 dbuf.at[b, j], sld[b])

        def wait_loads(k, b):
            base = base_of(k)
            for j in range(G):
                pltpu.make_async_copy(src_hbm.at[pl.ds(base + j * C, C)],
                                      sbuf.at[b, j], sld[b]).wait()
                pltpu.make_async_copy(dst_hbm.at[pl.ds(base + j * C, C)],
                                      dbuf.at[b, j], sld[b]).wait()

        def copy_ibuf(b):
            for j in range(G):
                def mk(m, _):
                    ibuf[b, j, pl.ds(m * 16, 16)] = dbuf[b, j, pl.ds(m * 16, 16)]
                    return 0
                lax.fori_loop(0, C // 16, mk, 0)

        def issue_gathers(b):
            for j in range(G):
                pltpu.async_copy(tbl_hbm.at[sbuf.at[b, j]], rows.at[b, j], sg[b])

        def drain_gathers(b):
            for j in range(G):
                pltpu.make_async_copy(tbl_hbm.at[sbuf.at[b, j]],
                                      rows.at[b, j], sg[b]).wait()

        def issue_scatters(b):
            for j in range(G):
                pltpu.async_copy(rows.at[b, j], acc.at[ibuf.at[b, j]],
                                 ssc[b], add=True)

        def drain_scatters(b):
            for j in range(G):
                pltpu.make_async_copy(rows.at[b, j], acc.at[ibuf.at[b, j]],
                                      ssc[b]).wait()

        def halfstep(k, b):
            wait_loads(k, b)

            @pl.when(k >= 2)
            def _():
                drain_scatters(b)
            copy_ibuf(b)
            issue_gathers(b)

            @pl.when(k >= 1)
            def _():
                drain_gathers(1 - b)

            @pl.when(k + 1 < nk)
            def _():
                issue_loads(k + 1, 1 - b)

            @pl.when(k >= 1)
            def _():
                issue_scatters(1 - b)

        issue_loads(0, 0)

        def pair(p, _):
            k0 = p * 2
            halfstep(k0, 0)
            k1 = k0 + 1

            @pl.when(k1 < nk)
            def _():
                halfstep(k1, 1)
            return 0
        lax.fori_loop(0, (nk + 1) // 2, pair, 0)

        @pl.when(nk % 2 == 1)
        def _():
            drain_gathers(0)
            issue_scatters(0)
            drain_scatters(1)
            drain_scatters(0)

        @pl.when(nk % 2 == 0)
        def _():
            drain_gathers(1)
            issue_scatters(1)
            drain_scatters(0)
            drain_scatters(1)

    @pl.when(c == 0)
    def _():
        run(ga_hbm)

    @pl.when(c == 1)
    def _():
        run(gb_hbm)

    plsc.subcore_barrier()

    @pl.when(s < NS - 1)
    def _():
        @pl.when(c == 0)
        def _():
            pltpu.sync_copy(acc.at[pl.ds(s * 6400, 6400)],
                            outa_hbm.at[pl.ds(s * 6400, 6400)])

        @pl.when(c == 1)
        def _():
            pltpu.sync_copy(acc.at[pl.ds(s * 6400, 6400)],
                            outb_hbm.at[pl.ds(s * 6400, 6400)])

    @pl.when(s == NS - 1)
    def _():
        @pl.when(c == 0)
        def _():
            pltpu.sync_copy(acc.at[pl.ds(96000, 4000)],
                            outa_hbm.at[pl.ds(96000, 4000)])

        @pl.when(c == 1)
        def _():
            pltpu.sync_copy(acc.at[pl.ds(96000, 4000)],
                            outb_hbm.at[pl.ds(96000, 4000)])


EG = 4
EGC = EG * C
NEGROUP = E // EGC


@functools.partial(
    pl.kernel,
    out_type=jax.ShapeDtypeStruct((E, H), jnp.float32),
    mesh=_mesh,
    compiler_params=_sc_params,
    scratch_types=dict(
        sbuf=pltpu.VMEM((2, EG, C), jnp.int32),
        dbuf=pltpu.VMEM((2, EG, C), jnp.int32),
        prow=pltpu.VMEM((2, EG, C, H), jnp.float32),
        qrow=pltpu.VMEM((2, EG, C, H), jnp.float32),
        sem_ld0=pltpu.SemaphoreType.DMA,
        sem_ld1=pltpu.SemaphoreType.DMA,
        sem_g0=pltpu.SemaphoreType.DMA,
        sem_g1=pltpu.SemaphoreType.DMA,
        sem_st0=pltpu.SemaphoreType.DMA,
        sem_st1=pltpu.SemaphoreType.DMA,
    ),
)
def _edge_gather_kernel(src_hbm, dst_hbm, p_hbm, q_hbm, eo_hbm,
                        sbuf, dbuf, prow, qrow,
                        sem_ld0, sem_ld1, sem_g0, sem_g1, sem_st0, sem_st1):
    c = lax.axis_index("c")
    s = lax.axis_index("s")
    wid = s * NC + c
    sld = (sem_ld0, sem_ld1)
    sg = (sem_g0, sem_g1)
    sst = (sem_st0, sem_st1)
    nk = (NEGROUP - wid + NW - 1) // NW

    def base_of(k):
        return (wid + k * NW) * EGC

    def issue_loads(k, b):
        base = base_of(k)
        for j in range(EG):
            pltpu.async_copy(src_hbm.at[pl.ds(base + j * C, C)],
                             sbuf.at[b, j], sld[b])
            pltpu.async_copy(dst_hbm.at[pl.ds(base + j * C, C)],
                             dbuf.at[b, j], sld[b])

    def wait_loads(k, b):
        base = base_of(k)
        for j in range(EG):
            pltpu.make_async_copy(src_hbm.at[pl.ds(base + j * C, C)],
                                  sbuf.at[b, j], sld[b]).wait()
            pltpu.make_async_copy(dst_hbm.at[pl.ds(base + j * C, C)],
                                  dbuf.at[b, j], sld[b]).wait()

    def issue_gathers(b):
        for j in range(EG):
            pltpu.async_copy(p_hbm.at[sbuf.at[b, j]], prow.at[b, j], sg[b])
            pltpu.async_copy(q_hbm.at[dbuf.at[b, j]], qrow.at[b, j], sg[b])

    def drain_gathers(b):
        for j in range(EG):
            pltpu.make_async_copy(p_hbm.at[sbuf.at[b, j]],
                                  prow.at[b, j], sg[b]).wait()
            pltpu.make_async_copy(q_hbm.at[dbuf.at[b, j]],
                                  qrow.at[b, j], sg[b]).wait()

    def add_rows(b):
        for j in range(EG):
            def add(r, _):
                prow[b, j, r, pl.ds(0, 16)] = (
                    prow[b, j, r, pl.ds(0, 16)] + qrow[b, j, r, pl.ds(0, 16)])
                prow[b, j, r, pl.ds(16, 16)] = (
                    prow[b, j, r, pl.ds(16, 16)] + qrow[b, j, r, pl.ds(16, 16)])
                return 0
            lax.fori_loop(0, C, add, 0, unroll=4)

    def issue_stores(k, b):
        base = base_of(k)
        for j in range(EG):
            pltpu.async_copy(prow.at[b, j],
                             eo_hbm.at[pl.ds(base + j * C, C)], sst[b])

    def drain_stores(k, b):
        base = base_of(k)
        for j in range(EG):
            pltpu.make_async_copy(prow.at[b, j],
                                  eo_hbm.at[pl.ds(base + j * C, C)],
                                  sst[b]).wait()

    def halfstep(k, b):
        wait_loads(k, b)

        @pl.when(k >= 2)
        def _():
            drain_stores(k - 2, b)
        issue_gathers(b)

        @pl.when(k >= 1)
        def _():
            drain_gathers(1 - b)
            add_rows(1 - b)
            issue_stores(k - 1, 1 - b)

        @pl.when(k + 1 < nk)
        def _():
            issue_loads(k + 1, 1 - b)

    issue_loads(0, 0)

    def pair(p, _):
        k0 = p * 2
        halfstep(k0, 0)
        k1 = k0 + 1

        @pl.when(k1 < nk)
        def _():
            halfstep(k1, 1)
        return 0
    lax.fori_loop(0, (nk + 1) // 2, pair, 0)

    @pl.when(nk % 2 == 1)
    def _():
        drain_gathers(0)
        add_rows(0)
        issue_stores(nk - 1, 0)
        drain_stores(nk - 2, 1)
        drain_stores(nk - 1, 0)

    @pl.when(nk % 2 == 0)
    def _():
        drain_gathers(1)
        add_rows(1)
        issue_stores(nk - 1, 1)
        drain_stores(nk - 2, 0)
        drain_stores(nk - 1, 1)


_BN = 2048


def _mm(w, a):
    return lax.dot_general(w, a, (((1,), (0,)), ((), ())),
                           preferred_element_type=jnp.float32)


def _node_a_body(xt_ref, cnt0t_ref, cnt1t_ref, w1_ref, b1_ref, w2_ref,
                 b2_ref, wg1_ref, g1_ref, dinv_ref):
    h = jnp.maximum(_mm(w1_ref[...], xt_ref[...]) + b1_ref[...][:, None], 0.0)
    h = _mm(w2_ref[...], h) + b2_ref[...][:, None]
    deg = cnt0t_ref[0:1, :] + cnt1t_ref[0:1, :] + 1.0
    dinv = lax.rsqrt(deg)
    dinv_ref[...] = dinv
    g1_ref[...] = _mm(wg1_ref[...], h) * dinv


def _node_b_body(tmp1at_ref, tmp1bt_ref, g1_ref, dinv_ref, bg1_ref, wg2_ref,
                 g2_ref):
    dinv = dinv_ref[...]
    tmp1t = jnp.concatenate([tmp1at_ref[...], tmp1bt_ref[...]], axis=0)
    h1 = jnp.maximum(dinv * (tmp1t + g1_ref[...])
                     + bg1_ref[...][:, None], 0.0)
    g2_ref[...] = _mm(wg2_ref[...], h1) * dinv


def _node_c_body(tmp2at_ref, tmp2bt_ref, g2_ref, dinv_ref, bg2_ref, as_ref,
                 ad_ref, h_ref, p_ref, q_ref):
    tmp2t = jnp.concatenate([tmp2at_ref[...], tmp2bt_ref[...]], axis=0)
    h = jnp.maximum(dinv_ref[...] * (tmp2t + g2_ref[...])
                    + bg2_ref[...][:, None], 0.0)
    h_ref[...] = h
    p_ref[...] = _mm(as_ref[...], h)
    q_ref[...] = _mm(ad_ref[...], h)


_BE = 6400


def _edge_d_body(eot_ref, eat_ref, aattr_ref, be1_ref, we2_ref, be2_ref,
                 e_ref):
    t = eot_ref[...] + _mm(aattr_ref[...], eat_ref[...]) + be1_ref[...][:, None]
    t = jnp.maximum(t, 0.0)
    e_ref[...] = _mm(we2_ref[...], t) + be2_ref[...][:, None]


def _col_spec(rows, b):
    return pl.BlockSpec((rows, b), lambda i: (0, i))


def _full(shape):
    return pl.BlockSpec(shape, lambda i: tuple(0 for _ in shape))


def kernel(x, edge_index, edge_attr, W1, b1, W2, b2, Wg1, bg1, Wg2, bg2,
           We1, be1, We2, be2):
    src = edge_index[0]
    dst = edge_index[1]

    cnt0, cnt1 = _deg_kernel(dst)

    n_grid = pl.cdiv(N, _BN)

    g1t, dinv = pl.pallas_call(
        _node_a_body,
        grid=(n_grid,),
        in_specs=[_col_spec(2, _BN), _col_spec(16, _BN), _col_spec(16, _BN),
                  _full((H, 2)), _full((H,)), _full((H, H)), _full((H,)),
                  _full((H, H))],
        out_specs=[_col_spec(H, _BN), _col_spec(1, _BN)],
        out_shape=[jax.ShapeDtypeStruct((H, N), jnp.float32),
                   jax.ShapeDtypeStruct((1, N), jnp.float32)],
    )(x.T, cnt0.T, cnt1.T, W1, b1, W2, b2, Wg1)

    tmp1a, tmp1b = _scatter_kernel(src, dst,
                                   jnp.transpose(g1t[:16]),
                                   jnp.transpose(g1t[16:]))

    g2t = pl.pallas_call(
        _node_b_body,
        grid=(n_grid,),
        in_specs=[_col_spec(16, _BN), _col_spec(16, _BN), _col_spec(H, _BN),
                  _col_spec(1, _BN), _full((H,)), _full((H, H))],
        out_specs=_col_spec(H, _BN),
        out_shape=jax.ShapeDtypeStruct((H, N), jnp.float32),
    )(tmp1a.T, tmp1b.T, g1t, dinv, bg1, Wg2)

    tmp2a, tmp2b = _scatter_kernel(src, dst,
                                   jnp.transpose(g2t[:16]),
                                   jnp.transpose(g2t[16:]))

    As = We1[:, :H]
    Ad = We1[:, H:2 * H]
    Aattr = We1[:, 2 * H:]

    ht, Pt, Qt = pl.pallas_call(
        _node_c_body,
        grid=(n_grid,),
        in_specs=[_col_spec(16, _BN), _col_spec(16, _BN), _col_spec(H, _BN),
                  _col_spec(1, _BN), _full((H,)), _full((H, H)),
                  _full((H, H))],
        out_specs=[_col_spec(H, _BN)] * 3,
        out_shape=[jax.ShapeDtypeStruct((H, N), jnp.float32)] * 3,
    )(tmp2a.T, tmp2b.T, g2t, dinv, bg2, As, Ad)

    eo = _edge_gather_kernel(src, dst, Pt.T, Qt.T)

    et = pl.pallas_call(
        _edge_d_body,
        grid=(E // _BE,),
        in_specs=[_col_spec(H, _BE), _col_spec(3, _BE),
                  _full((H, 3)), _full((H,)), _full((H, H)), _full((H,))],
        out_specs=_col_spec(H, _BE),
        out_shape=jax.ShapeDtypeStruct((H, E), jnp.float32),
    )(eo.T, edge_attr.T, Aattr, be1, We2, be2)

    return ht.T, et.T
